# baseline (speedup 1.0000x reference)
import jax
import jax.numpy as jnp
from jax import lax
from jax.experimental import pallas as pl
from jax.experimental.pallas import tpu as pltpu
from jax.experimental.pallas import tpu_sc as plsc

_NC = 2
_NS = 16
_NW = _NC * _NS

_B = 16384 * 50
_D = 16
_BPW = _B // _NW
_CHUNK = 2560
_NCHUNKS = _BPW // _CHUNK


def _gather_body(idx_hbm, table_hbm, out_hbm, idx_v, rows_v, sem):
    wid = lax.axis_index("s") * _NC + lax.axis_index("c")
    base = wid * _BPW
    pltpu.sync_copy(idx_hbm.at[pl.ds(base, _BPW)], idx_v)

    def chunk(c, carry):
        off = c * _CHUNK
        pltpu.async_copy(
            table_hbm.at[idx_v.at[pl.ds(off, _CHUNK)]], rows_v, sem
        ).wait()
        pltpu.sync_copy(rows_v, out_hbm.at[pl.ds(base + off, _CHUNK)])
        return carry

    lax.fori_loop(0, _NCHUNKS, chunk, 0)


@jax.jit
def kernel(indices, table):
    flat_idx = indices.reshape(-1).astype(jnp.int32)
    mesh = plsc.VectorSubcoreMesh(core_axis_name="c", subcore_axis_name="s")
    out = pl.kernel(
        _gather_body,
        out_type=jax.ShapeDtypeStruct((_B, _D), jnp.float32),
        mesh=mesh,
        scratch_types=[
            pltpu.VMEM((_BPW,), jnp.int32),
            pltpu.VMEM((_CHUNK, _D), jnp.float32),
            pltpu.SemaphoreType.DMA,
        ],
        compiler_params=pltpu.CompilerParams(use_tc_tiling_on_sc=False),
    )(flat_idx, table)
    return out.reshape(indices.shape + (_D,))


# speedup vs baseline: 1.0038x; 1.0038x over previous
import jax
import jax.numpy as jnp
from jax import lax
from jax.experimental import pallas as pl
from jax.experimental.pallas import tpu as pltpu
from jax.experimental.pallas import tpu_sc as plsc

_NC = 2
_NS = 16
_NW = _NC * _NS

_B = 16384 * 50
_D = 16
_BPW = _B // _NW
_CHUNK = 1280
_NCHUNKS = _BPW // _CHUNK
_NBUF = 4
_AHEAD = 2


def _gather_body(idx_hbm, table_hbm, out_hbm, idx_v, *bufs_and_sems):
    rows = bufs_and_sems[:_NBUF]
    gsem = bufs_and_sems[_NBUF:2 * _NBUF]
    wsem = bufs_and_sems[2 * _NBUF:3 * _NBUF]

    wid = lax.axis_index("s") * _NC + lax.axis_index("c")
    base = wid * _BPW
    pltpu.sync_copy(idx_hbm.at[pl.ds(base, _BPW)], idx_v)

    def start_gather(g):
        b = g % _NBUF
        return pltpu.async_copy(
            table_hbm.at[idx_v.at[pl.ds(g * _CHUNK, _CHUNK)]], rows[b], gsem[b]
        )

    gathers = {}
    writes = {}
    for g in range(_AHEAD):
        gathers[g] = start_gather(g)

    for g in range(_NCHUNKS):
        b = g % _NBUF
        gathers.pop(g).wait()
        writes[g] = pltpu.async_copy(
            rows[b], out_hbm.at[pl.ds(base + g * _CHUNK, _CHUNK)], wsem[b]
        )
        ng = g + _AHEAD
        if ng < _NCHUNKS:
            pending = ng - _NBUF
            if pending >= 0:
                writes.pop(pending).wait()
            gathers[ng] = start_gather(ng)

    for g in sorted(writes):
        writes.pop(g).wait()


@jax.jit
def kernel(indices, table):
    flat_idx = indices.reshape(-1).astype(jnp.int32)
    mesh = plsc.VectorSubcoreMesh(core_axis_name="c", subcore_axis_name="s")
    scratch = [pltpu.VMEM((_BPW,), jnp.int32)]
    scratch += [pltpu.VMEM((_CHUNK, _D), jnp.float32) for _ in range(_NBUF)]
    scratch += [pltpu.SemaphoreType.DMA for _ in range(2 * _NBUF)]
    out = pl.kernel(
        _gather_body,
        out_type=jax.ShapeDtypeStruct((_B, _D), jnp.float32),
        mesh=mesh,
        scratch_types=scratch,
        compiler_params=pltpu.CompilerParams(use_tc_tiling_on_sc=False),
    )(flat_idx, table)
    return out.reshape(indices.shape + (_D,))


# speedup vs baseline: 1.7153x; 1.7089x over previous
import jax
import jax.numpy as jnp
from jax import lax
from jax.experimental import pallas as pl
from jax.experimental.pallas import tpu as pltpu
from jax.experimental.pallas import tpu_sc as plsc

_NC = 2
_NS = 16
_NW = _NC * _NS
_BATCH = 16384
_HIST = 50
_D = 16
_BPW = _BATCH // _NW
_NBC = _BPW // 128


def _body(idx_hbm, table_hbm, out_hbm, idx_v, sidx0, sidx1, rows0, rows1,
          outv0, outv1, gsem0, gsem1):
    wid = lax.axis_index("s") * _NC + lax.axis_index("c")
    b0 = wid * _BPW
    bc0 = wid * _NBC
    pltpu.sync_copy(idx_hbm.at[pl.ds(b0 * _HIST, _BPW * _HIST)], idx_v)
    lane = lax.iota(jnp.int32, 16)

    def extract(h, sidx):
        for k in range(_BPW // 16):
            pos = lane * _HIST + (k * 16 * _HIST + h)
            sidx[pl.ds(k * 16, 16)] = plsc.load_gather(idx_v, [pos])

    def start(sidx, rows, sem):
        return pltpu.async_copy(table_hbm.at[sidx], rows, sem)

    def wait(sidx, rows, sem):
        pltpu.make_async_copy(table_hbm.at[sidx], rows, sem).wait()

    def drain(h, rows, outv):
        for tr in range(2):
            for bcl in range(_NBC):
                for r in range(8):
                    e = jnp.full((16,), tr * 8 + r, jnp.int32)
                    for bk in range(8):
                        ridx = lane + (bcl * 128 + bk * 16)
                        outv[tr, bcl, r, pl.ds(bk * 16, 16)] = (
                            plsc.load_gather(rows, [ridx, e]))
        pltpu.sync_copy(outv.at[0], out_hbm.at[h, 0, pl.ds(bc0, _NBC)])
        pltpu.sync_copy(outv.at[1], out_hbm.at[h, 1, pl.ds(bc0, _NBC)])

    extract(0, sidx0)
    start(sidx0, rows0, gsem0)

    def group(g, carry):
        h0 = g * 2
        extract(h0 + 1, sidx1)
        start(sidx1, rows1, gsem1)
        wait(sidx0, rows0, gsem0)
        drain(h0, rows0, outv0)

        @pl.when(h0 + 2 < _HIST)
        def _():
            extract(h0 + 2, sidx0)
            start(sidx0, rows0, gsem0)

        wait(sidx1, rows1, gsem1)
        drain(h0 + 1, rows1, outv1)
        return carry

    lax.fori_loop(0, _HIST // 2, group, 0)


@jax.jit
def kernel(indices, table):
    flat_idx = indices.reshape(-1).astype(jnp.int32)
    mesh = plsc.VectorSubcoreMesh(core_axis_name="c", subcore_axis_name="s")
    scratch = [
        pltpu.VMEM((_BPW * _HIST,), jnp.int32),
        pltpu.VMEM((_BPW,), jnp.int32),
        pltpu.VMEM((_BPW,), jnp.int32),
        pltpu.VMEM((_BPW, _D), jnp.float32),
        pltpu.VMEM((_BPW, _D), jnp.float32),
        pltpu.VMEM((2, _NBC, 8, 128), jnp.float32),
        pltpu.VMEM((2, _NBC, 8, 128), jnp.float32),
        pltpu.SemaphoreType.DMA,
        pltpu.SemaphoreType.DMA,
    ]
    out5 = pl.kernel(
        _body,
        out_type=jax.ShapeDtypeStruct((_HIST, 2, 128, 8, 128), jnp.float32),
        mesh=mesh,
        scratch_types=scratch,
        compiler_params=pltpu.CompilerParams(
            use_tc_tiling_on_sc=False, needs_layout_passes=False),
    )(flat_idx, table)
    return out5.transpose(2, 4, 0, 1, 3).reshape(_BATCH, _HIST, _D)
